# baseline (speedup 1.0000x reference)
import functools

import jax
import jax.numpy as jnp
from jax import lax
from jax.experimental import pallas as pl
from jax.experimental.pallas import tpu as pltpu
from jax.experimental.pallas import tpu_sc as plsc

_NC = 2
_NS = 16
_NW = _NC * _NS
_UNIT = 128
_K = 24
_BLK = _K * _UNIT


def _round_up(a, b):
    return (a + b - 1) // b * b


@functools.lru_cache(maxsize=None)
def _sc_scatter(n_tbl, e_rows, t):
    rows_per_w = e_rows // _NW
    nblocks = rows_per_w // _K
    rows_per_tile = n_tbl // _NS
    mesh = plsc.VectorSubcoreMesh(core_axis_name="c", subcore_axis_name="s")

    def body(table, src, dst, zeros2d, zeros1d, agg_out, deg_out,
             acc, deg_acc, src_v, dst_v, rows_v, ones_v, gsem, ssem):
        c = lax.axis_index("c")
        s = lax.axis_index("s")
        w = c * _NS + s
        r0 = s * rows_per_tile

        pltpu.sync_copy(zeros2d.at[pl.ds(r0, rows_per_tile)],
                        acc.at[pl.ds(r0, rows_per_tile)])
        pltpu.sync_copy(zeros1d.at[pl.ds(r0, rows_per_tile)],
                        deg_acc.at[pl.ds(r0, rows_per_tile)])
        for i in range(_UNIT // 16):
            ones_v[pl.ds(i * 16, 16)] = jnp.ones((16,), jnp.float32)
        plsc.subcore_barrier()

        row_base = w * rows_per_w

        @pl.loop(0, nblocks)
        def _block(b):
            rb = row_base + b * _K
            pltpu.sync_copy(src.at[pl.ds(rb, _K)], src_v)
            pltpu.sync_copy(dst.at[pl.ds(rb, _K)], dst_v)
            gds = []
            for j in range(_K):
                gds.append(pltpu.async_copy(table.at[src_v.at[j]],
                                            rows_v.at[j], gsem))
            dds = []
            for j in range(_K):
                dds.append(pltpu.async_copy(ones_v, deg_acc.at[src_v.at[j]],
                                            ssem, add=True))
            for d in gds:
                d.wait()
            sds = []
            for j in range(_K):
                sds.append(pltpu.async_copy(rows_v.at[j],
                                            acc.at[dst_v.at[j]],
                                            ssem, add=True))
            for d in dds:
                d.wait()
            for d in sds:
                d.wait()

        plsc.subcore_barrier()
        pltpu.sync_copy(acc.at[pl.ds(r0, rows_per_tile)],
                        agg_out.at[c, pl.ds(r0, rows_per_tile)])
        pltpu.sync_copy(deg_acc.at[pl.ds(r0, rows_per_tile)],
                        deg_out.at[pl.ds(c * n_tbl + r0, rows_per_tile)])

    return pl.kernel(
        body,
        out_type=(
            jax.ShapeDtypeStruct((_NC, n_tbl, t), jnp.float32),
            jax.ShapeDtypeStruct((_NC * n_tbl,), jnp.float32),
        ),
        mesh=mesh,
        scratch_types=[
            pltpu.VMEM_SHARED((n_tbl, t), jnp.float32),
            pltpu.VMEM_SHARED((n_tbl,), jnp.float32),
            pltpu.VMEM((_K, _UNIT), jnp.int32),
            pltpu.VMEM((_K, _UNIT), jnp.int32),
            pltpu.VMEM((_K, _UNIT, t), jnp.float32),
            pltpu.VMEM((_UNIT,), jnp.float32),
            pltpu.SemaphoreType.DMA,
            pltpu.SemaphoreType.DMA,
        ],
        compiler_params=pltpu.CompilerParams(use_tc_tiling_on_sc=False),
    )


def _combine_body(x_ref, deg_ref, agg_ref, a1_ref, a2_ref, g_ref, b_ref,
                  wb_ref, pf_ref, out_ref):
    deg = jnp.maximum(deg_ref[0:1, :] + deg_ref[1:2, :], 1.0)
    ld = jnp.log(deg)
    dp = jax.nn.sigmoid(g_ref[0, 0])
    sw = jnp.exp(a1_ref[0, 0])
    nw = sw * jnp.tanh(a2_ref[0, 0])
    agg = agg_ref[0] + agg_ref[1]
    wr = x_ref[...] * jnp.exp(dp * ld)
    post = jnp.where(pf_ref[0, 0] != 0,
                     jnp.exp((dp - 1.0) * ld), jnp.ones_like(ld))
    outv = sw * wr + nw * agg * post
    out_ref[...] = outv + jnp.where(wb_ref[0, 0] != 0, b_ref[0, 0], 0.0)


def _combine(x, deg2, agg_t, a1, a2, g, b, wb, post_flag):
    t, n = x.shape
    return pl.pallas_call(
        _combine_body,
        out_shape=jax.ShapeDtypeStruct((t, n), jnp.float32),
    )(x, deg2, agg_t, a1, a2, g, b, wb, post_flag)


def _prescale_body(xp_ref, deg_ref, g_ref, zp_ref):
    dp = jax.nn.sigmoid(g_ref[0, 0])
    deg = jnp.maximum(deg_ref[0:1, :] + deg_ref[1:2, :], 1.0)
    f = jnp.exp((dp - 1.0) * jnp.log(deg))
    zp_ref[...] = xp_ref[...] * f


def kernel(x, edge_index, alpha1, alpha2, gamma, bias, transpose, with_bias):
    t, n = x.shape
    e = edge_index.shape[1]
    n_tbl = _round_up(n + 32, _NS * _UNIT)
    e_pad = _round_up(e, _NW * _BLK)
    e_rows = e_pad // _UNIT

    xt = jnp.pad(jnp.transpose(x), ((0, n_tbl - n), (0, 0)))
    pad_len = e_pad - e
    pad_idx = n + (jnp.arange(pad_len, dtype=jnp.int32) % 32)
    src = jnp.concatenate([edge_index[0], pad_idx]).reshape(e_rows, _UNIT)
    dst = jnp.concatenate([edge_index[1], pad_idx]).reshape(e_rows, _UNIT)
    z2 = jnp.zeros((n_tbl, t), jnp.float32)
    z1 = jnp.zeros((n_tbl,), jnp.float32)

    scatter = _sc_scatter(n_tbl, e_rows, t)
    agg2, deg_flat = scatter(xt, src, dst, z2, z1)
    deg2 = deg_flat.reshape(_NC, n_tbl)
    deg2n = deg2[:, :n]
    wb = jnp.asarray(with_bias, jnp.int32).reshape(1, 1)

    def branch_plain(_):
        agg_t = jnp.swapaxes(agg2[:, :n, :], 1, 2)
        pf = jnp.ones((1, 1), jnp.int32)
        return _combine(x, deg2n, agg_t, alpha1, alpha2, gamma, bias, wb, pf)

    def branch_transpose(_):
        xp = jnp.pad(x, ((0, 0), (0, n_tbl - n)))
        zp = pl.pallas_call(
            _prescale_body,
            out_shape=jax.ShapeDtypeStruct((t, n_tbl), jnp.float32),
        )(xp, deg2, gamma)
        zt = jnp.transpose(zp)
        agg2b, _ = scatter(zt, src, dst, z2, z1)
        agg_t = jnp.swapaxes(agg2b[:, :n, :], 1, 2)
        pf = jnp.zeros((1, 1), jnp.int32)
        return _combine(x, deg2n, agg_t, alpha1, alpha2, gamma, bias, wb, pf)

    return lax.cond(jnp.asarray(transpose) == 0,
                    branch_plain, branch_transpose, operand=None)


# speedup vs baseline: 1.1627x; 1.1627x over previous
import functools

import jax
import jax.numpy as jnp
from jax import lax
from jax.experimental import pallas as pl
from jax.experimental.pallas import tpu as pltpu
from jax.experimental.pallas import tpu_sc as plsc

_NC = 2
_NS = 16
_NW = _NC * _NS
_UNIT = 128
_K = 8
_BLK = _K * _UNIT


def _round_up(a, b):
    return (a + b - 1) // b * b


@functools.lru_cache(maxsize=None)
def _sc_scatter(n_tbl, e_rows, t):
    rows_per_w = e_rows // _NW
    nblocks = rows_per_w // _K
    rows_per_tile = n_tbl // _NS
    mesh = plsc.VectorSubcoreMesh(core_axis_name="c", subcore_axis_name="s")

    def body(table, src, dst, zeros2d, zeros1d, agg_out, deg_out,
             acc, deg_acc, src_v, dst_v, rows_v, ones_v, gsem, ssem):
        c = lax.axis_index("c")
        s = lax.axis_index("s")
        w = c * _NS + s
        r0 = s * rows_per_tile

        pltpu.sync_copy(zeros2d.at[pl.ds(r0, rows_per_tile)],
                        acc.at[pl.ds(r0, rows_per_tile)])
        pltpu.sync_copy(zeros1d.at[pl.ds(r0, rows_per_tile)],
                        deg_acc.at[pl.ds(r0, rows_per_tile)])
        for i in range(_UNIT // 16):
            ones_v[pl.ds(i * 16, 16)] = jnp.ones((16,), jnp.float32)
        plsc.subcore_barrier()

        row_base = w * rows_per_w

        @pl.loop(0, nblocks)
        def _block(b):
            rb = row_base + b * _K
            pltpu.sync_copy(src.at[pl.ds(rb, _K)], src_v)
            pltpu.sync_copy(dst.at[pl.ds(rb, _K)], dst_v)
            gds = []
            for j in range(_K):
                gds.append(pltpu.async_copy(table.at[src_v.at[j]],
                                            rows_v.at[j], gsem))
            dds = []
            for j in range(_K):
                dds.append(pltpu.async_copy(ones_v, deg_acc.at[src_v.at[j]],
                                            ssem, add=True))
            for d in gds:
                d.wait()
            sds = []
            for j in range(_K):
                sds.append(pltpu.async_copy(rows_v.at[j],
                                            acc.at[dst_v.at[j]],
                                            ssem, add=True))
            for d in dds:
                d.wait()
            for d in sds:
                d.wait()

        plsc.subcore_barrier()
        pltpu.sync_copy(acc.at[pl.ds(r0, rows_per_tile)],
                        agg_out.at[c, pl.ds(r0, rows_per_tile)])
        pltpu.sync_copy(deg_acc.at[pl.ds(r0, rows_per_tile)],
                        deg_out.at[pl.ds(c * n_tbl + r0, rows_per_tile)])

    return pl.kernel(
        body,
        out_type=(
            jax.ShapeDtypeStruct((_NC, n_tbl, t), jnp.float32),
            jax.ShapeDtypeStruct((_NC * n_tbl,), jnp.float32),
        ),
        mesh=mesh,
        scratch_types=[
            pltpu.VMEM_SHARED((n_tbl, t), jnp.float32),
            pltpu.VMEM_SHARED((n_tbl,), jnp.float32),
            pltpu.VMEM((_K, _UNIT), jnp.int32),
            pltpu.VMEM((_K, _UNIT), jnp.int32),
            pltpu.VMEM((_K, _UNIT, t), jnp.float32),
            pltpu.VMEM((_UNIT,), jnp.float32),
            pltpu.SemaphoreType.DMA,
            pltpu.SemaphoreType.DMA,
        ],
        compiler_params=pltpu.CompilerParams(use_tc_tiling_on_sc=False),
    )


def _combine_body(x_ref, deg_ref, agg_ref, a1_ref, a2_ref, g_ref, b_ref,
                  wb_ref, pf_ref, out_ref):
    deg = jnp.maximum(deg_ref[0:1, :] + deg_ref[1:2, :], 1.0)
    ld = jnp.log(deg)
    dp = jax.nn.sigmoid(g_ref[0, 0])
    sw = jnp.exp(a1_ref[0, 0])
    nw = sw * jnp.tanh(a2_ref[0, 0])
    agg = agg_ref[0] + agg_ref[1]
    wr = x_ref[...] * jnp.exp(dp * ld)
    post = jnp.where(pf_ref[0, 0] != 0,
                     jnp.exp((dp - 1.0) * ld), jnp.ones_like(ld))
    outv = sw * wr + nw * agg * post
    out_ref[...] = outv + jnp.where(wb_ref[0, 0] != 0, b_ref[0, 0], 0.0)


def _combine(x, deg2, agg_t, a1, a2, g, b, wb, post_flag):
    t, n = x.shape
    return pl.pallas_call(
        _combine_body,
        out_shape=jax.ShapeDtypeStruct((t, n), jnp.float32),
    )(x, deg2, agg_t, a1, a2, g, b, wb, post_flag)


def _prescale_body(xp_ref, deg_ref, g_ref, zp_ref):
    dp = jax.nn.sigmoid(g_ref[0, 0])
    deg = jnp.maximum(deg_ref[0:1, :] + deg_ref[1:2, :], 1.0)
    f = jnp.exp((dp - 1.0) * jnp.log(deg))
    zp_ref[...] = xp_ref[...] * f


def kernel(x, edge_index, alpha1, alpha2, gamma, bias, transpose, with_bias):
    t, n = x.shape
    e = edge_index.shape[1]
    n_tbl = _round_up(n + 32, _NS * _UNIT)
    e_pad = _round_up(e, _NW * _BLK)
    e_rows = e_pad // _UNIT

    xt = jnp.pad(jnp.transpose(x), ((0, n_tbl - n), (0, 0)))
    pad_len = e_pad - e
    pad_idx = n + (jnp.arange(pad_len, dtype=jnp.int32) % 32)
    src = jnp.concatenate([edge_index[0], pad_idx]).reshape(e_rows, _UNIT)
    dst = jnp.concatenate([edge_index[1], pad_idx]).reshape(e_rows, _UNIT)
    z2 = jnp.zeros((n_tbl, t), jnp.float32)
    z1 = jnp.zeros((n_tbl,), jnp.float32)

    scatter = _sc_scatter(n_tbl, e_rows, t)
    agg2, deg_flat = scatter(xt, src, dst, z2, z1)
    deg2 = deg_flat.reshape(_NC, n_tbl)
    deg2n = deg2[:, :n]
    wb = jnp.asarray(with_bias, jnp.int32).reshape(1, 1)

    def branch_plain(_):
        agg_t = jnp.swapaxes(agg2[:, :n, :], 1, 2)
        pf = jnp.ones((1, 1), jnp.int32)
        return _combine(x, deg2n, agg_t, alpha1, alpha2, gamma, bias, wb, pf)

    def branch_transpose(_):
        xp = jnp.pad(x, ((0, 0), (0, n_tbl - n)))
        zp = pl.pallas_call(
            _prescale_body,
            out_shape=jax.ShapeDtypeStruct((t, n_tbl), jnp.float32),
        )(xp, deg2, gamma)
        zt = jnp.transpose(zp)
        agg2b, _ = scatter(zt, src, dst, z2, z1)
        agg_t = jnp.swapaxes(agg2b[:, :n, :], 1, 2)
        pf = jnp.zeros((1, 1), jnp.int32)
        return _combine(x, deg2n, agg_t, alpha1, alpha2, gamma, bias, wb, pf)

    return lax.cond(jnp.asarray(transpose) == 0,
                    branch_plain, branch_transpose, operand=None)
